# baseline (speedup 1.0000x reference)
import functools

import jax
import jax.numpy as jnp
from jax import lax
from jax.experimental import pallas as pl
from jax.experimental.pallas import tpu as pltpu
from jax.experimental.pallas import tpu_sc as plsc

_LANES = 16
_NC = 2
_NS = 16
_NW = _NC * _NS
_K = 64


def _matmul_block(x_ref, w_ref, b_ref, out_ref):
    out_ref[...] = lax.dot_general(
        x_ref[...], w_ref[...], (((1,), (1,)), ((), ())),
        preferred_element_type=jnp.float32) + b_ref[...]


def _add_block(p_ref, out_ref):
    out_ref[...] = p_ref[0] + p_ref[1]


def _make_scatter(n_pad, d, ep):
    per_tile = ep // _NW
    blocks = per_tile // _K
    rows_per_tile = n_pad // _NS
    zc = _K
    mesh = plsc.VectorSubcoreMesh(core_axis_name="c", subcore_axis_name="s")

    nsub = d // _LANES
    k8 = _K * nsub
    sub_rows = n_pad * nsub
    sub_per_tile = sub_rows // _NS
    assert blocks % 4 == 0 and blocks >= 4

    @functools.partial(
        pl.kernel,
        out_type=jax.ShapeDtypeStruct((_NC, n_pad, d), jnp.float32),
        mesh=mesh,
        compiler_params=pltpu.CompilerParams(use_tc_tiling_on_sc=False),
        scratch_types=(
            [pltpu.VMEM((_K,), jnp.int32)] * 4
            + [pltpu.VMEM((k8,), jnp.int32)] * 4
            + [pltpu.VMEM((_K * _LANES,), jnp.float32)] * 4
            + [pltpu.VMEM((_K, d), jnp.float32)] * 2
            + [pltpu.VMEM((k8, _LANES), jnp.float32)] * 2
            + [pltpu.VMEM_SHARED((sub_rows, _LANES), jnp.float32)]
            + [pltpu.SemaphoreType.DMA] * 8
        ),
    )
    def scatter(support_hbm, src_hbm, dst8_hbm, w_hbm, out_hbm,
                sv0, sv1, sv2, sv3, dv0, dv1, dv2, dv3, wv0, wv1, wv2, wv3,
                rv0, rv1, qv0, qv1, acc,
                st0, st1, st2, st3, g0, g1, sc0, sc1):
        srcs = [sv0, sv1, sv2, sv3]
        dsts = [dv0, dv1, dv2, dv3]
        ws = [wv0, wv1, wv2, wv3]
        rows = [rv0, rv1]
        rows8 = [qv0, qv1]
        sts = [st0, st1, st2, st3]
        gs = [g0, g1]
        scs = [sc0, sc1]

        cid = lax.axis_index("c")
        sid = lax.axis_index("s")
        ebase = (cid * _NS + sid) * per_tile
        rbase = sid * sub_per_tile

        def stage_refs(blk, s4):
            eb = pl.multiple_of(ebase + blk * _K, _K)
            return (
                (src_hbm.at[pl.ds(eb, _K)], srcs[s4]),
                (dst8_hbm.at[pl.ds(pl.multiple_of(eb * nsub, k8), k8)],
                 dsts[s4]),
                (w_hbm.at[pl.ds(pl.multiple_of(eb * _LANES, _K * _LANES),
                                _K * _LANES)], ws[s4]),
            )

        def stage_start(blk, s4):
            for a, b in stage_refs(blk, s4):
                pltpu.make_async_copy(a, b, sts[s4]).start()

        def stage_wait(blk, s4):
            for a, b in stage_refs(blk, s4):
                pltpu.make_async_copy(a, b, sts[s4]).wait()

        def zero_row(r, carry):
            qv0[r, :] = jnp.zeros((_LANES,), jnp.float32)
            return carry
        lax.fori_loop(0, k8, zero_row, 0)
        for i in range(sub_per_tile // k8):
            pltpu.sync_copy(qv0.at[pl.ds(0, k8)],
                            acc.at[pl.ds(rbase + i * k8, k8)])
        plsc.subcore_barrier()

        stage_start(0, 0)
        stage_start(1, 1)
        stage_wait(0, 0)
        pltpu.make_async_copy(support_hbm.at[srcs[0]], rows[0], gs[0]).start()

        def group_body(grp, carry):
            for u in range(4):
                blk = grp * 4 + u
                s2, s4 = u % 2, u
                o2, o4 = (u + 1) % 2, (u + 1) % 4

                @pl.when(blk + 1 < blocks)
                def _():
                    stage_wait(blk + 1, o4)
                    pltpu.make_async_copy(
                        support_hbm.at[srcs[o4]], rows[o2], gs[o2]).start()

                pltpu.make_async_copy(
                    support_hbm.at[srcs[s4]], rows[s2], gs[s2]).wait()

                @pl.when(blk >= 2)
                def _():
                    pltpu.make_async_copy(
                        rows8[s2], acc.at[dsts[s4]], scs[s2]).wait()

                @pl.when(blk + 2 < blocks)
                def _():
                    stage_start(blk + 2, (u + 2) % 4)

                def edge_body(e, c2):
                    wv = ws[s4][pl.ds(pl.multiple_of(e * _LANES, _LANES),
                                      _LANES)]
                    e8 = e * nsub
                    for j in range(nsub):
                        rows8[s2][e8 + j, :] = (
                            rows[s2][e, pl.ds(j * _LANES, _LANES)] * wv)
                    return c2
                lax.fori_loop(0, _K, edge_body, 0)

                pltpu.async_copy(rows8[s2], acc.at[dsts[s4]], scs[s2],
                                 add=True)
            return carry
        lax.fori_loop(0, blocks // 4, group_body, 0)

        pltpu.make_async_copy(rows8[0], acc.at[dsts[2]], scs[0]).wait()
        pltpu.make_async_copy(rows8[1], acc.at[dsts[3]], scs[1]).wait()

        plsc.subcore_barrier()
        rowbase = sid * (n_pad // _NS)
        for i in range(sub_per_tile // k8):
            pltpu.sync_copy(acc.at[pl.ds(rbase + i * k8, k8)], qv0)

            def repack_row(r, carry):
                r8 = r * nsub
                for j in range(nsub):
                    rv0[r, pl.ds(j * _LANES, _LANES)] = qv0[r8 + j, :]
                return carry
            lax.fori_loop(0, _K, repack_row, 0)
            pltpu.sync_copy(
                rv0, out_hbm.at[cid, pl.ds(rowbase + i * _K, _K)])

    return scatter


def kernel(input, edge_index, edge_weight, W, b):
    n, d_in = input.shape
    d_out = W.shape[0]
    e = edge_weight.shape[0]
    assert d_in % _LANES == 0 and d_out % _LANES == 0
    n_pad = -(-n // (_NS * _K)) * (_NS * _K)

    rb = 1000
    grid = (n // rb,)
    support = pl.pallas_call(
        _matmul_block,
        grid=grid,
        in_specs=[pl.BlockSpec((rb, d_in), lambda i: (i, 0)),
                  pl.BlockSpec((d_out, d_in), lambda i: (0, 0)),
                  pl.BlockSpec((1, d_out), lambda i: (0, 0))],
        out_specs=pl.BlockSpec((rb, d_out), lambda i: (i, 0)),
        out_shape=jax.ShapeDtypeStruct((n, d_out), jnp.float32),
    )(input, W, b.reshape(1, d_out))

    chunk = _NW * _K * 4
    ep = ((e + chunk - 1) // chunk) * chunk
    pad = ep - e
    src = jnp.concatenate(
        [edge_index[1].astype(jnp.int32), jnp.zeros((pad,), jnp.int32)])
    dst = jnp.concatenate(
        [edge_index[0].astype(jnp.int32), jnp.zeros((pad,), jnp.int32)])
    w = jnp.concatenate(
        [edge_weight.astype(jnp.float32), jnp.zeros((pad,), jnp.float32)])
    w = jnp.broadcast_to(w[:, None], (ep, _LANES)).reshape(ep * _LANES)
    nsub = d_out // _LANES
    dst8 = (dst[:, None] * nsub + jnp.arange(nsub, dtype=jnp.int32)
            ).reshape(ep * nsub)

    partials = _make_scatter(n_pad, d_out, ep)(support, src, dst8, w)

    out = pl.pallas_call(
        _add_block,
        grid=grid,
        in_specs=[pl.BlockSpec((_NC, rb, d_out), lambda i: (0, i, 0))],
        out_specs=pl.BlockSpec((rb, d_out), lambda i: (i, 0)),
        out_shape=jax.ShapeDtypeStruct((n, d_out), jnp.float32),
    )(partials)
    return out


# speedup vs baseline: 1.2582x; 1.2582x over previous
import functools

import jax
import jax.numpy as jnp
import numpy as np
from jax import lax
from jax.experimental import pallas as pl
from jax.experimental.pallas import tpu as pltpu
from jax.experimental.pallas import tpu_sc as plsc

_LANES = 16
_L2 = 32
_NC = 2
_NS = 16
_NW = _NC * _NS
_K = 128


def _matmul_block(x_ref, w_ref, b_ref, out_ref):
    out_ref[...] = (lax.dot_general(
        x_ref[...], w_ref[...], (((1,), (1,)), ((), ())),
        preferred_element_type=jnp.float32) + b_ref[...]
    ).astype(jnp.bfloat16)


def _add_block(p_ref, out_ref):
    out_ref[...] = (p_ref[0].astype(jnp.float32)
                    + p_ref[1].astype(jnp.float32))


def _feature_perm(d):
    perm = np.empty((d,), dtype=np.int32)
    for j in range(d // _L2):
        for t in range(_LANES):
            perm[_L2 * j + 2 * t] = _L2 * j + t
            perm[_L2 * j + 2 * t + 1] = _L2 * j + _LANES + t
    return perm


def _make_scatter(n_pad, d, ep):
    per_tile = ep // _NW
    blocks = per_tile // _K
    nsub = d // _L2
    k8 = _K * nsub
    sub_rows = n_pad * nsub
    sub_per_tile = sub_rows // _NS
    mesh = plsc.VectorSubcoreMesh(core_axis_name="c", subcore_axis_name="s")

    @functools.partial(
        pl.kernel,
        out_type=jax.ShapeDtypeStruct((_NC, n_pad, d), jnp.bfloat16),
        mesh=mesh,
        compiler_params=pltpu.CompilerParams(use_tc_tiling_on_sc=False,
                                             needs_layout_passes=False),
        scratch_types=[
            pltpu.VMEM((_K,), jnp.int32),
            pltpu.VMEM((k8,), jnp.int32),
            pltpu.VMEM((_K * _LANES,), jnp.float32),
            pltpu.VMEM((_K, d), jnp.bfloat16),
            pltpu.VMEM((k8, _L2), jnp.bfloat16),
            pltpu.VMEM_SHARED((sub_rows, _L2), jnp.bfloat16),
            pltpu.SemaphoreType.DMA,
        ],
    )
    def scatter(support_hbm, src_hbm, dst4_hbm, w_hbm, out_hbm,
                src_v, dst4_v, w_v, rows_v, rows8_v, acc, sem):
        cid = lax.axis_index("c")
        sid = lax.axis_index("s")
        ebase = (cid * _NS + sid) * per_tile
        rbase = sid * sub_per_tile

        def zero_row(r, carry):
            rows8_v[r, :] = jnp.zeros((_L2,), jnp.bfloat16)
            return carry
        lax.fori_loop(0, k8, zero_row, 0)
        for i in range(sub_per_tile // k8):
            pltpu.sync_copy(rows8_v.at[pl.ds(0, k8)],
                            acc.at[pl.ds(rbase + i * k8, k8)])
        plsc.subcore_barrier()

        def block_body(blk, carry):
            eb = pl.multiple_of(ebase + blk * _K, _K)
            pltpu.sync_copy(src_hbm.at[pl.ds(eb, _K)], src_v)
            pltpu.sync_copy(
                dst4_hbm.at[pl.ds(pl.multiple_of(eb * nsub, k8), k8)], dst4_v)
            pltpu.sync_copy(
                w_hbm.at[pl.ds(pl.multiple_of(eb * _LANES, _K * _LANES),
                               _K * _LANES)], w_v)
            pltpu.async_copy(support_hbm.at[src_v], rows_v, sem).wait()

            def edge_body(e, c2):
                wv = w_v[pl.ds(pl.multiple_of(e * _LANES, _LANES), _LANES)]
                e4 = e * nsub
                for j in range(nsub):
                    v = rows_v[e, pl.ds(j * _L2, _L2)]
                    a, b2 = plsc.unpack(v, format=plsc.PackFormat.INTERLEAVED)
                    rows8_v[e4 + j, :] = plsc.pack(
                        a * wv, b2 * wv, format=plsc.PackFormat.INTERLEAVED)
                return c2
            lax.fori_loop(0, _K, edge_body, 0)

            pltpu.sync_copy(rows8_v, acc.at[dst4_v], add=True)
            return carry
        lax.fori_loop(0, blocks, block_body, 0)

        plsc.subcore_barrier()
        rowbase = sid * (n_pad // _NS)
        rows_per_chunk = k8 // nsub
        for i in range(sub_per_tile // k8):
            pltpu.sync_copy(acc.at[pl.ds(rbase + i * k8, k8)], rows8_v)

            def repack_row(r, carry):
                r4 = r * nsub
                for j in range(nsub):
                    rows_v[r, pl.ds(j * _L2, _L2)] = rows8_v[r4 + j, :]
                return carry
            lax.fori_loop(0, rows_per_chunk, repack_row, 0)
            pltpu.sync_copy(
                rows_v.at[pl.ds(0, rows_per_chunk)],
                out_hbm.at[cid, pl.ds(rowbase + i * rows_per_chunk,
                                      rows_per_chunk)])

    return scatter


def kernel(input, edge_index, edge_weight, W, b):
    n, d_in = input.shape
    d_out = W.shape[0]
    e = edge_weight.shape[0]
    assert d_in % _LANES == 0 and d_out % _L2 == 0
    n_pad = -(-n // (_NS * _K)) * (_NS * _K)

    perm = _feature_perm(d_out)
    inv = np.argsort(perm)
    W_p = W[jnp.asarray(perm)]
    b_p = b[jnp.asarray(perm)]

    rb = 1000
    grid = (n // rb,)
    support = pl.pallas_call(
        _matmul_block,
        grid=grid,
        in_specs=[pl.BlockSpec((rb, d_in), lambda i: (i, 0)),
                  pl.BlockSpec((d_out, d_in), lambda i: (0, 0)),
                  pl.BlockSpec((1, d_out), lambda i: (0, 0))],
        out_specs=pl.BlockSpec((rb, d_out), lambda i: (i, 0)),
        out_shape=jax.ShapeDtypeStruct((n, d_out), jnp.bfloat16),
    )(input, W_p, b_p.reshape(1, d_out))

    chunk = _NW * _K
    ep = ((e + chunk - 1) // chunk) * chunk
    pad = ep - e
    src = jnp.concatenate(
        [edge_index[1].astype(jnp.int32), jnp.zeros((pad,), jnp.int32)])
    dst = jnp.concatenate(
        [edge_index[0].astype(jnp.int32), jnp.zeros((pad,), jnp.int32)])
    w = jnp.concatenate(
        [edge_weight.astype(jnp.float32), jnp.zeros((pad,), jnp.float32)])
    w = jnp.broadcast_to(w[:, None], (ep, _LANES)).reshape(ep * _LANES)
    nsub = d_out // _L2
    dst4 = (dst[:, None] * nsub + jnp.arange(nsub, dtype=jnp.int32)
            ).reshape(ep * nsub)

    partials = _make_scatter(n_pad, d_out, ep)(support, src, dst4, w)

    out_p = pl.pallas_call(
        _add_block,
        grid=grid,
        in_specs=[pl.BlockSpec((_NC, rb, d_out), lambda i: (0, i, 0))],
        out_specs=pl.BlockSpec((rb, d_out), lambda i: (i, 0)),
        out_shape=jax.ShapeDtypeStruct((n, d_out), jnp.float32),
    )(partials)
    return out_p[:, jnp.asarray(inv)]
